# baseline (speedup 1.0000x reference)
import functools

import jax
import jax.numpy as jnp
from jax import lax
from jax.experimental import pallas as pl
from jax.experimental.pallas import tpu as pltpu
from jax.experimental.pallas import tpu_sc as plsc

N = 10000
D = 128
C = 40
CP = 48
E = 320000

NC = 2
NS = 16
NW = NC * NS
LANES = 16

B = 128
NBLK = E // B
BPW = NBLK // NW
BPR = 6
ROUNDS = BPW // BPR
LAST = ROUNDS - 1
RPS = N // NS
ZCH = 125
WB = 624


def _mm_body(nf_ref, w_ref, b_ref, o_ref):
    o_ref[...] = (
        jnp.dot(nf_ref[...], w_ref[...],
                preferred_element_type=jnp.float32,
                precision=lax.Precision.HIGHEST)
        + b_ref[...]
    )


def _project(nf, w_pad, b_pad):
    blk = 1000
    return pl.pallas_call(
        _mm_body,
        grid=(N // blk,),
        in_specs=[
            pl.BlockSpec((blk, D), lambda i: (i, 0)),
            pl.BlockSpec((D, CP), lambda i: (0, 0)),
            pl.BlockSpec((1, CP), lambda i: (0, 0)),
        ],
        out_specs=pl.BlockSpec((blk, CP), lambda i: (i, 0)),
        out_shape=jax.ShapeDtypeStruct((N, CP), jnp.float32),
    )(nf, w_pad, b_pad)


def _combine_body(p_ref, o_ref):
    s = p_ref[0] + p_ref[1]
    o_ref[...] = s[:, :C]


def _combine(partials):
    blk = 1000
    return pl.pallas_call(
        _combine_body,
        grid=(N // blk,),
        in_specs=[pl.BlockSpec((NC, blk, CP), lambda i: (0, i, 0))],
        out_specs=pl.BlockSpec((blk, C), lambda i: (i, 0)),
        out_shape=jax.ShapeDtypeStruct((N, C), jnp.float32),
    )(partials)


@functools.partial(
    pl.kernel,
    out_type=jax.ShapeDtypeStruct((NC, N, CP), jnp.float32),
    mesh=plsc.VectorSubcoreMesh(core_axis_name="c", subcore_axis_name="s"),
    scratch_types=[
        pltpu.VMEM_SHARED((N, CP), jnp.float32),
        pltpu.VMEM((2, BPR, B), jnp.int32),
        pltpu.VMEM((2, BPR, B), jnp.int32),
        pltpu.VMEM((2, BPR, B), jnp.int32),
        pltpu.VMEM((2, BPR, B), jnp.float32),
        pltpu.VMEM((2, BPR, B, CP), jnp.float32),
        pltpu.SemaphoreType.DMA,
        pltpu.SemaphoreType.DMA,
        pltpu.SemaphoreType.DMA,
        pltpu.SemaphoreType.DMA,
        pltpu.SemaphoreType.DMA,
    ],
    compiler_params=pltpu.CompilerParams(use_tc_tiling_on_sc=False),
)
def _sc_smooth(x_hbm, col_hbm, row_hbm, val_hbm, out_hbm,
               acc, col_v, row_v, srow_v, val_v, rows_v,
               esem, gsem0, gsem1, ssem0, ssem1):
    cid = lax.axis_index("c")
    sid = lax.axis_index("s")
    wid = sid * NC + cid
    wbase = wid * BPW
    gsem = (gsem0, gsem1)
    ssem = (ssem0, ssem1)

    def edges(r, b, start):
        blk = wbase + r * BPR
        for hbm, vm in ((col_hbm, col_v), (row_hbm, row_v), (val_hbm, val_v)):
            d = pltpu.make_async_copy(hbm.at[pl.ds(blk, BPR)], vm.at[b], esem)
            d.start() if start else d.wait()

    def gathers(r, b, start):
        @pl.loop(0, BPR)
        def _(j):
            d = pltpu.make_async_copy(
                x_hbm.at[col_v.at[b, j]], rows_v.at[b, j], gsem[b])
            d.start() if start else d.wait()

    def scale(b):
        @pl.loop(0, BPR)
        def _(j):
            @plsc.parallel_loop(0, B // LANES, unroll=4)
            def _(g):
                e0 = g * LANES
                srow_v[b, j, pl.ds(e0, LANES)] = row_v[b, j, pl.ds(e0, LANES)]
                vv = val_v[b, j, pl.ds(e0, LANES)]
                for l in range(LANES):
                    vs = vv[l]
                    for c3 in range(CP // LANES):
                        sl = pl.ds(c3 * LANES, LANES)
                        rows_v[b, j, e0 + l, sl] = rows_v[b, j, e0 + l, sl] * vs

    def scatters(r, b, start):
        @pl.loop(0, BPR)
        def _(j):
            if start:
                pltpu.async_copy(rows_v.at[b, j], acc.at[srow_v.at[b, j]],
                                 ssem[b], add=True)
            else:
                pltpu.make_async_copy(
                    rows_v.at[b, j], acc.at[srow_v.at[b, j]], ssem[b]).wait()

    edges(0, 0, True)

    zeros16 = jnp.zeros((LANES,), jnp.float32)

    @pl.loop(0, ZCH)
    def _(e):
        for c3 in range(CP // LANES):
            rows_v[0, 0, e, pl.ds(c3 * LANES, LANES)] = zeros16

    @pl.loop(0, RPS // ZCH)
    def _(r):
        pltpu.sync_copy(rows_v.at[0, 0, pl.ds(0, ZCH)],
                        acc.at[pl.ds(sid * RPS + r * ZCH, ZCH)])

    edges(0, 0, False)
    edges(1, 1, True)
    gathers(0, 0, True)
    plsc.subcore_barrier()

    @pl.loop(0, LAST, step=2)
    def _(i):
        for b in (0, 1):
            r = i + b
            edges(r + 1, 1 - b, False)

            @pl.when(r >= 1)
            def _():
                scatters(r - 1, 1 - b, False)

            gathers(r + 1, 1 - b, True)
            gathers(r, b, False)
            scale(b)

            @pl.when(r + 2 <= LAST)
            def _():
                edges(r + 2, b, True)

            scatters(r, b, True)

    scatters(LAST - 1, 1, False)
    gathers(LAST, 0, False)
    scale(0)
    scatters(LAST, 0, True)
    scatters(LAST, 0, False)

    @pl.when(wid < NBLK - BPW * NW)
    def _():
        tb = BPW * NW + wid
        for hbm, vm in ((col_hbm, col_v), (row_hbm, row_v), (val_hbm, val_v)):
            pltpu.sync_copy(hbm.at[pl.ds(tb, 1)], vm.at[0, pl.ds(0, 1)])
        pltpu.async_copy(x_hbm.at[col_v.at[0, 0]], rows_v.at[0, 0],
                         gsem0).wait()

        @pl.loop(0, B // LANES)
        def _(g):
            e0 = g * LANES
            vv = val_v[0, 0, pl.ds(e0, LANES)]
            for l in range(LANES):
                vs = vv[l]
                for c3 in range(CP // LANES):
                    sl = pl.ds(c3 * LANES, LANES)
                    rows_v[0, 0, e0 + l, sl] = rows_v[0, 0, e0 + l, sl] * vs

        pltpu.sync_copy(rows_v.at[0, 0], acc.at[row_v.at[0, 0]], add=True)

    plsc.subcore_barrier()

    pltpu.sync_copy(acc.at[pl.ds(sid * WB, WB)],
                    out_hbm.at[cid, pl.ds(sid * WB, WB)])

    @pl.when(sid < (N - NS * WB) // 8)
    def _():
        off = NS * WB + sid * 8
        pltpu.sync_copy(acc.at[pl.ds(off, 8)],
                        out_hbm.at[cid, pl.ds(off, 8)])


def kernel(node_features, edge_index, edge_vals, ids, W, b):
    del ids
    w_pad = jnp.zeros((D, CP), jnp.float32).at[:, :C].set(W)
    b_pad = jnp.zeros((1, CP), jnp.float32).at[0, :C].set(b)

    x_pad = _project(node_features, w_pad, b_pad)

    row2d = edge_index[0].reshape(NBLK, B)
    col2d = edge_index[1].reshape(NBLK, B)
    val2d = edge_vals.reshape(NBLK, B)

    partials = _sc_smooth(x_pad, col2d, row2d, val2d)
    return _combine(partials)


# speedup vs baseline: 1.0942x; 1.0942x over previous
import functools

import jax
import jax.numpy as jnp
from jax import lax
from jax.experimental import pallas as pl
from jax.experimental.pallas import tpu as pltpu
from jax.experimental.pallas import tpu_sc as plsc

N = 10000
D = 128
C = 40
CP = 48
E = 320000

NC = 2
NS = 16
NW = NC * NS
LANES = 16

B = 128
NBLK = E // B
BPW = NBLK // NW
BPR = 6
ROUNDS = BPW // BPR
LAST = ROUNDS - 1
RPS = N // NS
ZCH = 125
WB = 624


def _mm_body(nf_ref, w_ref, b_ref, o_ref):
    o_ref[...] = (
        jnp.dot(nf_ref[...], w_ref[...],
                preferred_element_type=jnp.float32,
                precision=lax.Precision.HIGHEST)
        + b_ref[...]
    )


def _project(nf, w_pad, b_pad):
    blk = 1000
    return pl.pallas_call(
        _mm_body,
        grid=(N // blk,),
        in_specs=[
            pl.BlockSpec((blk, D), lambda i: (i, 0)),
            pl.BlockSpec((D, CP), lambda i: (0, 0)),
            pl.BlockSpec((1, CP), lambda i: (0, 0)),
        ],
        out_specs=pl.BlockSpec((blk, CP), lambda i: (i, 0)),
        out_shape=jax.ShapeDtypeStruct((N, CP), jnp.float32),
    )(nf, w_pad, b_pad)


def _combine_body(p_ref, o_ref):
    s = p_ref[0] + p_ref[1]
    o_ref[...] = s[:, :C]


def _combine(partials):
    blk = 1000
    return pl.pallas_call(
        _combine_body,
        grid=(N // blk,),
        in_specs=[pl.BlockSpec((NC, blk, CP), lambda i: (0, i, 0))],
        out_specs=pl.BlockSpec((blk, C), lambda i: (i, 0)),
        out_shape=jax.ShapeDtypeStruct((N, C), jnp.float32),
    )(partials)


@functools.partial(
    pl.kernel,
    out_type=jax.ShapeDtypeStruct((NC, N, CP), jnp.float32),
    mesh=plsc.VectorSubcoreMesh(core_axis_name="c", subcore_axis_name="s"),
    scratch_types=[
        pltpu.VMEM_SHARED((N, CP), jnp.float32),
        pltpu.VMEM((2, BPR, B), jnp.int32),
        pltpu.VMEM((2, BPR, B), jnp.int32),
        pltpu.VMEM((2, BPR, B), jnp.int32),
        pltpu.VMEM((2, BPR, B), jnp.float32),
        pltpu.VMEM((2, BPR, B, CP), jnp.float32),
        pltpu.SemaphoreType.DMA,
        pltpu.SemaphoreType.DMA,
        pltpu.SemaphoreType.DMA,
        pltpu.SemaphoreType.DMA,
        pltpu.SemaphoreType.DMA,
    ],
    compiler_params=pltpu.CompilerParams(use_tc_tiling_on_sc=False),
)
def _sc_smooth(x_hbm, col_hbm, row_hbm, val_hbm, out_hbm,
               acc, col_v, row_v, srow_v, val_v, rows_v,
               esem, gsem0, gsem1, ssem0, ssem1):
    cid = lax.axis_index("c")
    sid = lax.axis_index("s")
    wid = sid * NC + cid
    wbase = wid * BPW
    gsem = (gsem0, gsem1)
    ssem = (ssem0, ssem1)

    def edges(r, b, start):
        blk = wbase + r * BPR
        for hbm, vm in ((col_hbm, col_v), (row_hbm, row_v), (val_hbm, val_v)):
            d = pltpu.make_async_copy(hbm.at[pl.ds(blk, BPR)], vm.at[b], esem)
            d.start() if start else d.wait()

    def gathers(r, b, start):
        @pl.loop(0, BPR)
        def _(j):
            d = pltpu.make_async_copy(
                x_hbm.at[col_v.at[b, j]], rows_v.at[b, j], gsem[b])
            d.start() if start else d.wait()

    def scale(b):
        @pl.loop(0, BPR)
        def _(j):
            @plsc.parallel_loop(0, B // LANES, unroll=2)
            def _(g):
                e0 = g * LANES
                srow_v[b, j, pl.ds(e0, LANES)] = row_v[b, j, pl.ds(e0, LANES)]
                vv = val_v[b, j, pl.ds(e0, LANES)]
                for l in range(LANES):
                    vs = vv[l]
                    for c3 in range(CP // LANES):
                        sl = pl.ds(c3 * LANES, LANES)
                        rows_v[b, j, e0 + l, sl] = rows_v[b, j, e0 + l, sl] * vs

    def scatters(r, b, start):
        @pl.loop(0, BPR)
        def _(j):
            if start:
                pltpu.async_copy(rows_v.at[b, j], acc.at[srow_v.at[b, j]],
                                 ssem[b], add=True)
            else:
                pltpu.make_async_copy(
                    rows_v.at[b, j], acc.at[srow_v.at[b, j]], ssem[b]).wait()

    edges(0, 0, True)

    zeros16 = jnp.zeros((LANES,), jnp.float32)

    @pl.loop(0, ZCH)
    def _(e):
        for c3 in range(CP // LANES):
            rows_v[0, 0, e, pl.ds(c3 * LANES, LANES)] = zeros16

    @pl.loop(0, RPS // ZCH)
    def _(r):
        pltpu.sync_copy(rows_v.at[0, 0, pl.ds(0, ZCH)],
                        acc.at[pl.ds(sid * RPS + r * ZCH, ZCH)])

    edges(0, 0, False)
    edges(1, 1, True)
    gathers(0, 0, True)
    plsc.subcore_barrier()

    @pl.loop(0, LAST, step=2)
    def _(i):
        for b in (0, 1):
            r = i + b
            edges(r + 1, 1 - b, False)

            @pl.when(r >= 1)
            def _():
                scatters(r - 1, 1 - b, False)

            gathers(r + 1, 1 - b, True)
            gathers(r, b, False)
            scale(b)

            @pl.when(r + 2 <= LAST)
            def _():
                edges(r + 2, b, True)

            scatters(r, b, True)

    scatters(LAST - 1, 1, False)
    gathers(LAST, 0, False)
    scale(0)
    scatters(LAST, 0, True)
    scatters(LAST, 0, False)

    @pl.when(wid < NBLK - BPW * NW)
    def _():
        tb = BPW * NW + wid
        for hbm, vm in ((col_hbm, col_v), (row_hbm, row_v), (val_hbm, val_v)):
            pltpu.sync_copy(hbm.at[pl.ds(tb, 1)], vm.at[0, pl.ds(0, 1)])
        pltpu.async_copy(x_hbm.at[col_v.at[0, 0]], rows_v.at[0, 0],
                         gsem0).wait()

        @pl.loop(0, B // LANES)
        def _(g):
            e0 = g * LANES
            vv = val_v[0, 0, pl.ds(e0, LANES)]
            for l in range(LANES):
                vs = vv[l]
                for c3 in range(CP // LANES):
                    sl = pl.ds(c3 * LANES, LANES)
                    rows_v[0, 0, e0 + l, sl] = rows_v[0, 0, e0 + l, sl] * vs

        pltpu.sync_copy(rows_v.at[0, 0], acc.at[row_v.at[0, 0]], add=True)

    plsc.subcore_barrier()

    pltpu.sync_copy(acc.at[pl.ds(sid * WB, WB)],
                    out_hbm.at[cid, pl.ds(sid * WB, WB)])

    @pl.when(sid < (N - NS * WB) // 8)
    def _():
        off = NS * WB + sid * 8
        pltpu.sync_copy(acc.at[pl.ds(off, 8)],
                        out_hbm.at[cid, pl.ds(off, 8)])


def kernel(node_features, edge_index, edge_vals, ids, W, b):
    del ids
    w_pad = jnp.zeros((D, CP), jnp.float32).at[:, :C].set(W)
    b_pad = jnp.zeros((1, CP), jnp.float32).at[0, :C].set(b)

    x_pad = _project(node_features, w_pad, b_pad)

    row2d = edge_index[0].reshape(NBLK, B)
    col2d = edge_index[1].reshape(NBLK, B)
    val2d = edge_vals.reshape(NBLK, B)

    partials = _sc_smooth(x_pad, col2d, row2d, val2d)
    return _combine(partials)


# speedup vs baseline: 1.2205x; 1.1154x over previous
import functools

import jax
import jax.numpy as jnp
from jax import lax
from jax.experimental import pallas as pl
from jax.experimental.pallas import tpu as pltpu
from jax.experimental.pallas import tpu_sc as plsc

N = 10000
D = 128
C = 40
CP = 48
E = 320000

NC = 2
NS = 16
NW = NC * NS
LANES = 16

B = 128
NBLK = E // B
BPW = NBLK // NW
BPR = 6
ROUNDS = BPW // BPR
LAST = ROUNDS - 1
RPS = N // NS
ZCH = 125
WB = 624


def _mm_body(nf_ref, w_ref, b_ref, o_ref):
    o_ref[...] = (
        jnp.dot(nf_ref[...], w_ref[...],
                preferred_element_type=jnp.float32)
        + b_ref[...]
    )


def _project(nf, w_pad, b_pad):
    blk = 1000
    return pl.pallas_call(
        _mm_body,
        grid=(N // blk,),
        in_specs=[
            pl.BlockSpec((blk, D), lambda i: (i, 0)),
            pl.BlockSpec((D, CP), lambda i: (0, 0)),
            pl.BlockSpec((1, CP), lambda i: (0, 0)),
        ],
        out_specs=pl.BlockSpec((blk, CP), lambda i: (i, 0)),
        out_shape=jax.ShapeDtypeStruct((N, CP), jnp.float32),
    )(nf, w_pad, b_pad)


def _combine_body(p_ref, o_ref):
    s = p_ref[0] + p_ref[1]
    o_ref[...] = s[:, :C]


def _combine(partials):
    blk = 1000
    return pl.pallas_call(
        _combine_body,
        grid=(N // blk,),
        in_specs=[pl.BlockSpec((NC, blk, CP), lambda i: (0, i, 0))],
        out_specs=pl.BlockSpec((blk, C), lambda i: (i, 0)),
        out_shape=jax.ShapeDtypeStruct((N, C), jnp.float32),
    )(partials)


@functools.partial(
    pl.kernel,
    out_type=jax.ShapeDtypeStruct((NC, N, CP), jnp.float32),
    mesh=plsc.VectorSubcoreMesh(core_axis_name="c", subcore_axis_name="s"),
    scratch_types=[
        pltpu.VMEM_SHARED((N, CP), jnp.float32),
        pltpu.VMEM((2, 2, BPR * B), jnp.int32),
        pltpu.VMEM((2, BPR, B), jnp.int32),
        pltpu.VMEM((2, BPR * B), jnp.float32),
        pltpu.VMEM((2, BPR, B, CP), jnp.float32),
        pltpu.SemaphoreType.DMA,
        pltpu.SemaphoreType.DMA,
        pltpu.SemaphoreType.DMA,
        pltpu.SemaphoreType.DMA,
        pltpu.SemaphoreType.DMA,
    ],
    compiler_params=pltpu.CompilerParams(use_tc_tiling_on_sc=False),
)
def _sc_smooth(x_hbm, eidx_hbm, val_hbm, out_hbm,
               acc, eidx_v, srow_v, val_v, rows_v,
               esem, gsem0, gsem1, ssem0, ssem1):
    cid = lax.axis_index("c")
    sid = lax.axis_index("s")
    wid = sid * NC + cid
    wbase = wid * BPW
    gsem = (gsem0, gsem1)
    ssem = (ssem0, ssem1)

    def edges(r, b, start):
        eoff = (wbase + r * BPR) * B
        d1 = pltpu.make_async_copy(
            eidx_hbm.at[pl.ds(0, 2), pl.ds(eoff, BPR * B)], eidx_v.at[b], esem)
        d2 = pltpu.make_async_copy(
            val_hbm.at[pl.ds(eoff, BPR * B)], val_v.at[b], esem)
        if start:
            d1.start()
            d2.start()
        else:
            d1.wait()
            d2.wait()

    def gathers(r, b, start):
        @pl.loop(0, BPR)
        def _(j):
            d = pltpu.make_async_copy(
                x_hbm.at[eidx_v.at[b, 1, pl.ds(j * B, B)]],
                rows_v.at[b, j], gsem[b])
            d.start() if start else d.wait()

    def scale(b):
        @pl.loop(0, BPR)
        def _(j):
            @plsc.parallel_loop(0, B // LANES, unroll=2)
            def _(g):
                e0 = g * LANES
                srow_v[b, j, pl.ds(e0, LANES)] = (
                    eidx_v[b, 0, pl.ds(j * B + e0, LANES)])
                vv = val_v[b, pl.ds(j * B + e0, LANES)]
                for l in range(LANES):
                    vs = vv[l]
                    for c3 in range(CP // LANES):
                        sl = pl.ds(c3 * LANES, LANES)
                        rows_v[b, j, e0 + l, sl] = rows_v[b, j, e0 + l, sl] * vs

    def scatters(r, b, start):
        @pl.loop(0, BPR)
        def _(j):
            if start:
                pltpu.async_copy(rows_v.at[b, j], acc.at[srow_v.at[b, j]],
                                 ssem[b], add=True)
            else:
                pltpu.make_async_copy(
                    rows_v.at[b, j], acc.at[srow_v.at[b, j]], ssem[b]).wait()

    edges(0, 0, True)

    zeros16 = jnp.zeros((LANES,), jnp.float32)

    @pl.loop(0, ZCH)
    def _(e):
        for c3 in range(CP // LANES):
            rows_v[0, 0, e, pl.ds(c3 * LANES, LANES)] = zeros16

    @pl.loop(0, RPS // ZCH)
    def _(r):
        pltpu.sync_copy(rows_v.at[0, 0, pl.ds(0, ZCH)],
                        acc.at[pl.ds(sid * RPS + r * ZCH, ZCH)])

    edges(0, 0, False)
    edges(1, 1, True)
    gathers(0, 0, True)
    plsc.subcore_barrier()

    @pl.loop(0, LAST, step=2)
    def _(i):
        for b in (0, 1):
            r = i + b
            edges(r + 1, 1 - b, False)

            @pl.when(r >= 1)
            def _():
                scatters(r - 1, 1 - b, False)

            gathers(r + 1, 1 - b, True)
            gathers(r, b, False)
            scale(b)

            @pl.when(r + 2 <= LAST)
            def _():
                edges(r + 2, b, True)

            scatters(r, b, True)

    scatters(LAST - 1, 1, False)
    gathers(LAST, 0, False)
    scale(0)
    scatters(LAST, 0, True)
    scatters(LAST, 0, False)

    @pl.when(wid < NBLK - BPW * NW)
    def _():
        toff = (BPW * NW + wid) * B
        pltpu.sync_copy(eidx_hbm.at[pl.ds(0, 2), pl.ds(toff, B)],
                        eidx_v.at[0, pl.ds(0, 2), pl.ds(0, B)])
        pltpu.sync_copy(val_hbm.at[pl.ds(toff, B)],
                        val_v.at[0, pl.ds(0, B)])
        pltpu.async_copy(x_hbm.at[eidx_v.at[0, 1, pl.ds(0, B)]],
                         rows_v.at[0, 0], gsem0).wait()

        @pl.loop(0, B // LANES)
        def _(g):
            e0 = g * LANES
            srow_v[0, 0, pl.ds(e0, LANES)] = eidx_v[0, 0, pl.ds(e0, LANES)]
            vv = val_v[0, pl.ds(e0, LANES)]
            for l in range(LANES):
                vs = vv[l]
                for c3 in range(CP // LANES):
                    sl = pl.ds(c3 * LANES, LANES)
                    rows_v[0, 0, e0 + l, sl] = rows_v[0, 0, e0 + l, sl] * vs

        pltpu.sync_copy(rows_v.at[0, 0], acc.at[srow_v.at[0, 0]], add=True)

    plsc.subcore_barrier()

    pltpu.sync_copy(acc.at[pl.ds(sid * WB, WB)],
                    out_hbm.at[cid, pl.ds(sid * WB, WB)])

    @pl.when(sid < (N - NS * WB) // 8)
    def _():
        off = NS * WB + sid * 8
        pltpu.sync_copy(acc.at[pl.ds(off, 8)],
                        out_hbm.at[cid, pl.ds(off, 8)])


def kernel(node_features, edge_index, edge_vals, ids, W, b):
    del ids
    w_pad = jnp.zeros((D, CP), jnp.float32).at[:, :C].set(W)
    b_pad = jnp.zeros((1, CP), jnp.float32).at[0, :C].set(b)

    x_pad = _project(node_features, w_pad, b_pad)

    partials = _sc_smooth(x_pad, edge_index, edge_vals)
    return _combine(partials)
